# baseline (speedup 1.0000x reference)
import jax
import jax.numpy as jnp
from jax.experimental import pallas as pl
from jax.experimental.pallas import tpu as pltpu

_B, _N, _M, _C = 8, 2048, 2048, 128
_BN = 1024
_CHM = 512
_EPS_INV = 1e8


def _body(t_ref, b_ref, z_ref, x1_ref, x2_ref, cos_ref, mask_ref, s_ref, c_ref):
    nb = pl.program_id(1)
    first = nb == 0

    x1 = x1_ref[0]
    x2 = x2_ref[0]
    log2e = 1.4426950408889634
    tl2 = t_ref[0, 0] * log2e
    bl2 = b_ref[0, 0] * log2e

    r1 = jnp.minimum(
        jax.lax.rsqrt(jnp.sum(x1 * x1, axis=1, keepdims=True)), 1e4)
    ones = jnp.ones((1, _C), dtype=jnp.float32)
    s2 = jax.lax.dot_general(ones, x2 * x2, (((1,), (1,)), ((), ())),
                             preferred_element_type=jnp.float32)
    r2 = jnp.minimum(jax.lax.rsqrt(s2), 1e4)

    gi = jax.lax.broadcasted_iota(jnp.int32, (_BN // 16, _BN), 0)
    ni = jax.lax.broadcasted_iota(jnp.int32, (_BN // 16, _BN), 1)
    j = ni - 16 * gi
    w_pack = jnp.where((j >= 0) & (j < 16), jnp.exp2(j.astype(jnp.float32)), 0.0)

    for k in range(_M // _CHM):
        sl = slice(k * _CHM, (k + 1) * _CHM)
        dc = jax.lax.dot_general(x1, x2[sl, :], (((1,), (1,)), ((), ())),
                                 preferred_element_type=jnp.float32)
        cos = (dc * r1) * r2[:, sl]
        zc = z_ref[0, :, sl]
        zf = zc.astype(jnp.float32)
        u = bl2 - tl2 * cos
        e = jnp.exp2(zf * u)
        nll2 = jnp.log2(1.0 + e)
        zf2 = zf * zf

        cos_ref[0, :, sl] = cos
        packed = jax.lax.dot_general(w_pack, zf2, (((1,), (0,)), ((), ())),
                                     preferred_element_type=jnp.float32)
        mask_ref[0, :, sl] = jnp.round(packed).astype(jnp.int32)

        colsum = jnp.sum(nll2 * zf2, axis=0, keepdims=True)
        colcnt = jnp.sum(zf2, axis=0, keepdims=True)
        s_ref[0, :, sl] = jnp.where(first, colsum, s_ref[0, :, sl] + colsum)
        c_ref[0, :, sl] = jnp.where(first, colcnt, c_ref[0, :, sl] + colcnt)


def kernel(z, x1, x2, t, b):
    t2 = jnp.reshape(t, (1, 1))
    b2 = jnp.reshape(b, (1, 1))

    grid = (_B, _N // _BN)
    cos, mask, s_part, c_part = pl.pallas_call(
        _body,
        grid=grid,
        in_specs=[
            pl.BlockSpec(memory_space=pltpu.SMEM),
            pl.BlockSpec(memory_space=pltpu.SMEM),
            pl.BlockSpec((1, _BN, _M), lambda bb, nb: (bb, nb, 0)),
            pl.BlockSpec((1, _BN, _C), lambda bb, nb: (bb, nb, 0)),
            pl.BlockSpec((1, _M, _C), lambda bb, nb: (bb, 0, 0)),
        ],
        out_specs=(
            pl.BlockSpec((1, _BN, _M), lambda bb, nb: (bb, nb, 0)),
            pl.BlockSpec((1, _BN // 16, _M), lambda bb, nb: (bb, nb, 0)),
            pl.BlockSpec((1, 1, _M), lambda bb, nb: (bb, 0, 0)),
            pl.BlockSpec((1, 1, _M), lambda bb, nb: (bb, 0, 0)),
        ),
        out_shape=(
            jax.ShapeDtypeStruct((_B, _N, _M), jnp.float32),
            jax.ShapeDtypeStruct((_B, _N // 16, _M), jnp.int32),
            jax.ShapeDtypeStruct((_B, 1, _M), jnp.float32),
            jax.ShapeDtypeStruct((_B, 1, _M), jnp.float32),
        ),
        compiler_params=pltpu.CompilerParams(
            dimension_semantics=("parallel", "arbitrary"),
        ),
    )(t2, b2, z, x1, x2)

    loss = jnp.sum(s_part) * jnp.float32(0.6931471805599453) / jnp.sum(c_part)
    shifts = jnp.arange(16, dtype=jnp.int32).reshape(1, 1, 16, 1)
    bits = (mask[:, :, None, :] >> shifts) & 1
    return loss, cos, (bits != 0).reshape(_B, _N, _M)


# speedup vs baseline: 1.4598x; 1.4598x over previous
import jax
import jax.numpy as jnp
from jax.experimental import pallas as pl
from jax.experimental.pallas import tpu as pltpu

_B, _N, _M, _C = 8, 2048, 2048, 128
_BN = 1024
_CHM = 512
_EPS_INV = 1e8


def _body(t_ref, b_ref, z_ref, x1_ref, x2_ref, cos_ref, mask_ref, s_ref, c_ref):
    nb = pl.program_id(1)
    first = nb == 0

    x1 = x1_ref[0]
    x2 = x2_ref[0]
    log2e = 1.4426950408889634
    tl2 = t_ref[0, 0] * log2e
    bl2 = b_ref[0, 0] * log2e

    r1 = jnp.minimum(
        jax.lax.rsqrt(jnp.sum(x1 * x1, axis=1, keepdims=True)), 1e4)
    ones = jnp.ones((1, _C), dtype=jnp.float32)
    s2 = jax.lax.dot_general(ones, x2 * x2, (((1,), (1,)), ((), ())),
                             preferred_element_type=jnp.float32)
    r2 = jnp.minimum(jax.lax.rsqrt(s2), 1e4)

    for k in range(_M // _CHM):
        sl = slice(k * _CHM, (k + 1) * _CHM)
        dc = jax.lax.dot_general(x1, x2[sl, :], (((1,), (1,)), ((), ())),
                                 preferred_element_type=jnp.float32)
        cos = (dc * r1) * r2[:, sl]
        zc = z_ref[0, :, sl]
        zf = zc.astype(jnp.float32)
        u = bl2 - tl2 * cos
        e = jnp.exp2(zf * u)
        nll2 = jnp.log2(1.0 + e)
        zf2 = zf * zf

        cos_ref[0, :, sl] = cos
        mask_ref[0, :, sl] = (zc != 0).astype(jnp.int8)

        colsum = jnp.sum(nll2 * zf2, axis=0, keepdims=True)
        colcnt = jnp.sum(zf2, axis=0, keepdims=True)
        s_ref[0, :, sl] = jnp.where(first, colsum, s_ref[0, :, sl] + colsum)
        c_ref[0, :, sl] = jnp.where(first, colcnt, c_ref[0, :, sl] + colcnt)


def kernel(z, x1, x2, t, b):
    t2 = jnp.reshape(t, (1, 1))
    b2 = jnp.reshape(b, (1, 1))

    grid = (_B, _N // _BN)
    cos, mask, s_part, c_part = pl.pallas_call(
        _body,
        grid=grid,
        in_specs=[
            pl.BlockSpec(memory_space=pltpu.SMEM),
            pl.BlockSpec(memory_space=pltpu.SMEM),
            pl.BlockSpec((1, _BN, _M), lambda bb, nb: (bb, nb, 0)),
            pl.BlockSpec((1, _BN, _C), lambda bb, nb: (bb, nb, 0)),
            pl.BlockSpec((1, _M, _C), lambda bb, nb: (bb, 0, 0)),
        ],
        out_specs=(
            pl.BlockSpec((1, _BN, _M), lambda bb, nb: (bb, nb, 0)),
            pl.BlockSpec((1, _BN, _M), lambda bb, nb: (bb, nb, 0)),
            pl.BlockSpec((1, 1, _M), lambda bb, nb: (bb, 0, 0)),
            pl.BlockSpec((1, 1, _M), lambda bb, nb: (bb, 0, 0)),
        ),
        out_shape=(
            jax.ShapeDtypeStruct((_B, _N, _M), jnp.float32),
            jax.ShapeDtypeStruct((_B, _N, _M), jnp.int8),
            jax.ShapeDtypeStruct((_B, 1, _M), jnp.float32),
            jax.ShapeDtypeStruct((_B, 1, _M), jnp.float32),
        ),
        compiler_params=pltpu.CompilerParams(
            dimension_semantics=("parallel", "arbitrary"),
        ),
    )(t2, b2, z, x1, x2)

    loss = jnp.sum(s_part) * jnp.float32(0.6931471805599453) / jnp.sum(c_part)
    return loss, cos, mask.view(jnp.bool_)
